# baseline (speedup 1.0000x reference)
import functools

import jax
import jax.numpy as jnp
from jax import lax
from jax.experimental import pallas as pl
from jax.experimental.pallas import tpu as pltpu
from jax.experimental.pallas import tpu_sc as plsc

BEV_H, BEV_W, BEV_Z, D = 128, 128, 16, 128
M = BEV_H * BEV_W * BEV_Z
NIDX = M // 2

NC, NS, L = 2, 16, 16
NW = NC * NS
CHUNK = 128
ROWS_PER_W = NIDX // (NW * CHUNK)


def _mask_body(midx_hbm, uidx_hbm, mask_hbm, midx_v, uidx_v, ones_v, zeros_v,
               sem_m, sem_u):
    wid = lax.axis_index("s") * NC + lax.axis_index("c")
    base = wid * ROWS_PER_W
    pltpu.sync_copy(midx_hbm.at[pl.ds(base, ROWS_PER_W)], midx_v)
    pltpu.sync_copy(uidx_hbm.at[pl.ds(base, ROWS_PER_W)], uidx_v)
    for i in range(CHUNK // L):
        ones_v[pl.ds(i * L, L)] = jnp.ones((L,), jnp.float32)
        zeros_v[pl.ds(i * L, L)] = jnp.zeros((L,), jnp.float32)

    LAG = 8

    def body(j, carry):
        pltpu.async_copy(ones_v, mask_hbm.at[midx_v.at[j]], sem_m)
        pltpu.async_copy(zeros_v, mask_hbm.at[uidx_v.at[j]], sem_u)

        @pl.when(j >= LAG)
        def _():
            pltpu.make_async_copy(
                ones_v, mask_hbm.at[midx_v.at[j - LAG]], sem_m).wait()
            pltpu.make_async_copy(
                zeros_v, mask_hbm.at[uidx_v.at[j - LAG]], sem_u).wait()

        return carry

    lax.fori_loop(0, ROWS_PER_W, body, 0)
    for k in range(ROWS_PER_W - LAG, ROWS_PER_W):
        pltpu.make_async_copy(ones_v, mask_hbm.at[midx_v.at[k]], sem_m).wait()
        pltpu.make_async_copy(zeros_v, mask_hbm.at[uidx_v.at[k]], sem_u).wait()


def _build_mask(masked_idx, unmasked_idx):
    midx = masked_idx.reshape(NW * ROWS_PER_W, CHUNK)
    uidx = unmasked_idx.reshape(NW * ROWS_PER_W, CHUNK)
    mesh = plsc.VectorSubcoreMesh(core_axis_name="c", subcore_axis_name="s")
    fn = functools.partial(
        pl.kernel,
        mesh=mesh,
        out_type=jax.ShapeDtypeStruct((M,), jnp.float32),
        scratch_types=[
            pltpu.VMEM((ROWS_PER_W, CHUNK), jnp.int32),
            pltpu.VMEM((ROWS_PER_W, CHUNK), jnp.int32),
            pltpu.VMEM((CHUNK,), jnp.float32),
            pltpu.VMEM((CHUNK,), jnp.float32),
            pltpu.SemaphoreType.DMA,
            pltpu.SemaphoreType.DMA,
        ],
    )(_mask_body)
    return fn(midx, uidx)


def _fused_body(x_ref, m_ref, wsgb_ref, bsgb_ref, w1_ref, b1_ref, lns_ref,
                lnb_ref, w2_ref, b2_ref, o_ref):
    xt = x_ref[...].T
    sgb = jnp.dot(xt, wsgb_ref[...],
                  preferred_element_type=jnp.float32) + bsgb_ref[...]
    h = jnp.dot(xt, w1_ref[...],
                preferred_element_type=jnp.float32) + b1_ref[...]
    mu = jnp.mean(h, axis=-1, keepdims=True)
    dh = h - mu
    var = jnp.mean(dh * dh, axis=-1, keepdims=True)
    h = dh * lax.rsqrt(var + 1e-5) * lns_ref[...] + lnb_ref[...]
    h = jnp.where(h >= 0, h, 0.01 * h)
    prior = jnp.dot(h, w2_ref[...],
                    preferred_element_type=jnp.float32) + b2_ref[...]
    o_ref[...] = jnp.where(m_ref[...] > 0.5, prior, sgb)


def _fused(x3d, mask2d, W_sgb, b_sgb, W1, b1, ln_scale, ln_bias, W2, b2,
           block=2048):
    grid = (M // block,)
    zero2 = lambda i: (0, 0)
    return pl.pallas_call(
        _fused_body,
        grid=grid,
        in_specs=[
            pl.BlockSpec((D, block), lambda i: (0, i)),
            pl.BlockSpec((block, 1), lambda i: (i, 0)),
            pl.BlockSpec((D, D), zero2),
            pl.BlockSpec((1, D), zero2),
            pl.BlockSpec((D, D // 2), zero2),
            pl.BlockSpec((1, D // 2), zero2),
            pl.BlockSpec((1, D // 2), zero2),
            pl.BlockSpec((1, D // 2), zero2),
            pl.BlockSpec((D // 2, D), zero2),
            pl.BlockSpec((1, D), zero2),
        ],
        out_specs=pl.BlockSpec((block, D), lambda i: (i, 0)),
        out_shape=jax.ShapeDtypeStruct((M, D), jnp.float32),
        compiler_params=pltpu.CompilerParams(
            dimension_semantics=("arbitrary",)),
    )(x3d, mask2d, W_sgb, b_sgb.reshape(1, D), W1, b1.reshape(1, D // 2),
      ln_scale.reshape(1, D // 2), ln_bias.reshape(1, D // 2), W2,
      b2.reshape(1, D))


def kernel(x3d, unmasked_idx, masked_idx, W_sgb, b_sgb, W1, b1, ln_scale,
           ln_bias, W2, b2):
    mask = _build_mask(masked_idx, unmasked_idx)
    out = _fused(x3d, mask.reshape(M, 1), W_sgb, b_sgb, W1, b1, ln_scale,
                 ln_bias, W2, b2)
    return out.reshape(BEV_H, BEV_W, BEV_Z, D)


# speedup vs baseline: 1.5754x; 1.5754x over previous
import functools

import jax
import jax.numpy as jnp
from jax import lax
from jax.experimental import pallas as pl
from jax.experimental.pallas import tpu as pltpu
from jax.experimental.pallas import tpu_sc as plsc

BEV_H, BEV_W, BEV_Z, D = 128, 128, 16, 128
M = BEV_H * BEV_W * BEV_Z
NIDX = M // 2

NC, NS, L = 2, 16, 16
NW = NC * NS
CHUNK = 128
ROWS_PER_W = NIDX // (NW * CHUNK)


HALF = M // 2
PAD = HALF
SLICE = HALF // NS
N_CHUNKS = NIDX // (NS * CHUNK)


def _mask_body(midx_hbm, mask_hbm, idx_v, zeros_v, addr_v, val_v, shared):
    c = lax.axis_index("c")
    s = lax.axis_index("s")
    base = c * HALF

    pltpu.sync_copy(midx_hbm.at[pl.ds(s * N_CHUNKS, N_CHUNKS)], idx_v)
    def zbody(i, carry):
        zeros_v[pl.ds(i * L, L)] = jnp.zeros((L,), jnp.float32)
        return carry
    lax.fori_loop(0, SLICE // L, zbody, 0)
    pltpu.sync_copy(zeros_v, shared.at[pl.ds(s * SLICE, SLICE)])
    plsc.subcore_barrier()

    def body(j, carry):
        for i in range(CHUNK // L):
            idx16 = idx_v[j, pl.ds(i * L, L)]
            loc = idx16 - base
            inr = (loc >= 0) & (loc < HALF)
            addr_v[pl.ds(i * L, L)] = jnp.where(inr, loc, PAD)
            val_v[pl.ds(i * L, L)] = jnp.where(inr, 1.0, 0.0)
        pltpu.sync_copy(val_v, shared.at[addr_v], add=True)
        return carry

    lax.fori_loop(0, N_CHUNKS, body, 0)
    plsc.subcore_barrier()
    pltpu.sync_copy(shared.at[pl.ds(s * SLICE, SLICE)],
                    mask_hbm.at[pl.ds(base + s * SLICE, SLICE)])


def _build_mask(masked_idx, unmasked_idx):
    del unmasked_idx
    midx = masked_idx.reshape(NS * N_CHUNKS, CHUNK)
    mesh = plsc.VectorSubcoreMesh(core_axis_name="c", subcore_axis_name="s")
    fn = functools.partial(
        pl.kernel,
        mesh=mesh,
        out_type=jax.ShapeDtypeStruct((M,), jnp.float32),
        scratch_types=[
            pltpu.VMEM((N_CHUNKS, CHUNK), jnp.int32),
            pltpu.VMEM((SLICE,), jnp.float32),
            pltpu.VMEM((CHUNK,), jnp.int32),
            pltpu.VMEM((CHUNK,), jnp.float32),
            pltpu.VMEM_SHARED((HALF + CHUNK,), jnp.float32),
        ],
    )(_mask_body)
    return fn(midx)


def _fused_body(x_ref, m_ref, wsgb_ref, bsgb_ref, w1_ref, b1_ref, lns_ref,
                lnb_ref, w2_ref, b2_ref, o_ref):
    xt = x_ref[...].T
    sgb = jnp.dot(xt, wsgb_ref[...],
                  preferred_element_type=jnp.float32) + bsgb_ref[...]
    h = jnp.dot(xt, w1_ref[...],
                preferred_element_type=jnp.float32) + b1_ref[...]
    mu = jnp.mean(h, axis=-1, keepdims=True)
    dh = h - mu
    var = jnp.mean(dh * dh, axis=-1, keepdims=True)
    h = dh * lax.rsqrt(var + 1e-5) * lns_ref[...] + lnb_ref[...]
    h = jnp.where(h >= 0, h, 0.01 * h)
    prior = jnp.dot(h, w2_ref[...],
                    preferred_element_type=jnp.float32) + b2_ref[...]
    o_ref[...] = jnp.where(m_ref[...] > 0.5, prior, sgb)


def _fused(x3d, mask2d, W_sgb, b_sgb, W1, b1, ln_scale, ln_bias, W2, b2,
           block=2048):
    grid = (M // block,)
    zero2 = lambda i: (0, 0)
    return pl.pallas_call(
        _fused_body,
        grid=grid,
        in_specs=[
            pl.BlockSpec((D, block), lambda i: (0, i)),
            pl.BlockSpec((block, 1), lambda i: (i, 0)),
            pl.BlockSpec((D, D), zero2),
            pl.BlockSpec((1, D), zero2),
            pl.BlockSpec((D, D // 2), zero2),
            pl.BlockSpec((1, D // 2), zero2),
            pl.BlockSpec((1, D // 2), zero2),
            pl.BlockSpec((1, D // 2), zero2),
            pl.BlockSpec((D // 2, D), zero2),
            pl.BlockSpec((1, D), zero2),
        ],
        out_specs=pl.BlockSpec((block, D), lambda i: (i, 0)),
        out_shape=jax.ShapeDtypeStruct((M, D), jnp.float32),
        compiler_params=pltpu.CompilerParams(
            dimension_semantics=("arbitrary",)),
    )(x3d, mask2d, W_sgb, b_sgb.reshape(1, D), W1, b1.reshape(1, D // 2),
      ln_scale.reshape(1, D // 2), ln_bias.reshape(1, D // 2), W2,
      b2.reshape(1, D))


def kernel(x3d, unmasked_idx, masked_idx, W_sgb, b_sgb, W1, b1, ln_scale,
           ln_bias, W2, b2):
    mask = _build_mask(masked_idx, unmasked_idx)
    out = _fused(x3d, mask.reshape(M, 1), W_sgb, b_sgb, W1, b1, ln_scale,
                 ln_bias, W2, b2)
    return out.reshape(BEV_H, BEV_W, BEV_Z, D)


# speedup vs baseline: 1.5959x; 1.0130x over previous
import functools

import jax
import jax.numpy as jnp
from jax import lax
from jax.experimental import pallas as pl
from jax.experimental.pallas import tpu as pltpu
from jax.experimental.pallas import tpu_sc as plsc

BEV_H, BEV_W, BEV_Z, D = 128, 128, 16, 128
M = BEV_H * BEV_W * BEV_Z
NIDX = M // 2

NC, NS, L = 2, 16, 16
NW = NC * NS
CHUNK = 128
ROWS_PER_W = NIDX // (NW * CHUNK)


HALF = M // 2
PAD = HALF
SLICE = HALF // NS
N_CHUNKS = NIDX // (NS * CHUNK)


def _mask_body(midx_hbm, mask_hbm, idx_v, zeros_v, addr_v, val_v, shared):
    c = lax.axis_index("c")
    s = lax.axis_index("s")
    base = c * HALF

    pltpu.sync_copy(midx_hbm.at[pl.ds(s * N_CHUNKS, N_CHUNKS)], idx_v)
    def zbody(i, carry):
        zeros_v[pl.ds(i * L, L)] = jnp.zeros((L,), jnp.float32)
        return carry
    lax.fori_loop(0, SLICE // L, zbody, 0)
    pltpu.sync_copy(zeros_v, shared.at[pl.ds(s * SLICE, SLICE)])
    plsc.subcore_barrier()

    def body(j, carry):
        for i in range(CHUNK // L):
            idx16 = idx_v[j, pl.ds(i * L, L)]
            loc = idx16 - base
            inr = (loc >= 0) & (loc < HALF)
            addr_v[pl.ds(i * L, L)] = jnp.where(inr, loc, PAD)
            val_v[pl.ds(i * L, L)] = jnp.where(inr, 1.0, 0.0)
        pltpu.sync_copy(val_v, shared.at[addr_v], add=True)
        return carry

    lax.fori_loop(0, N_CHUNKS, body, 0)
    plsc.subcore_barrier()
    pltpu.sync_copy(shared.at[pl.ds(s * SLICE, SLICE)],
                    mask_hbm.at[pl.ds(base + s * SLICE, SLICE)])


def _build_mask(masked_idx, unmasked_idx):
    del unmasked_idx
    midx = masked_idx.reshape(NS * N_CHUNKS, CHUNK)
    mesh = plsc.VectorSubcoreMesh(core_axis_name="c", subcore_axis_name="s")
    fn = functools.partial(
        pl.kernel,
        mesh=mesh,
        out_type=jax.ShapeDtypeStruct((M,), jnp.float32),
        scratch_types=[
            pltpu.VMEM((N_CHUNKS, CHUNK), jnp.int32),
            pltpu.VMEM((SLICE,), jnp.float32),
            pltpu.VMEM((CHUNK,), jnp.int32),
            pltpu.VMEM((CHUNK,), jnp.float32),
            pltpu.VMEM_SHARED((HALF + CHUNK,), jnp.float32),
        ],
    )(_mask_body)
    return fn(midx)


def _fused_body(x_ref, m_ref, wsgb_ref, bsgb_ref, w1_ref, b1_ref, lns_ref,
                lnb_ref, w2_ref, b2_ref, o_ref):
    H = D // 2
    xt = x_ref[...].T
    sgb = jnp.dot(xt, wsgb_ref[...],
                  preferred_element_type=jnp.float32) + bsgb_ref[...]
    h = jnp.dot(xt, w1_ref[...],
                preferred_element_type=jnp.float32) + b1_ref[...]
    ones = jnp.ones((H, H), jnp.float32)
    mu = jnp.dot(h, ones, preferred_element_type=jnp.float32) * (1.0 / H)
    s2 = jnp.dot(h * h, ones, preferred_element_type=jnp.float32) * (1.0 / H)
    var = s2 - mu * mu
    hn = (h - mu) * lax.rsqrt(var + 1e-5) * lns_ref[...] + lnb_ref[...]
    hn = jnp.maximum(hn, 0.01 * hn)
    prior = jnp.dot(hn, w2_ref[...],
                    preferred_element_type=jnp.float32) + b2_ref[...]
    o_ref[...] = jnp.where(m_ref[...] > 0.5, prior, sgb)


def _fused(x3d, mask2d, W_sgb, b_sgb, W1, b1, ln_scale, ln_bias, W2, b2,
           block=2048):
    grid = (M // block,)
    zero2 = lambda i: (0, 0)
    return pl.pallas_call(
        _fused_body,
        grid=grid,
        in_specs=[
            pl.BlockSpec((D, block), lambda i: (0, i)),
            pl.BlockSpec((block, 1), lambda i: (i, 0)),
            pl.BlockSpec((D, D), zero2),
            pl.BlockSpec((1, D), zero2),
            pl.BlockSpec((D, D // 2), zero2),
            pl.BlockSpec((1, D // 2), zero2),
            pl.BlockSpec((1, D // 2), zero2),
            pl.BlockSpec((1, D // 2), zero2),
            pl.BlockSpec((D // 2, D), zero2),
            pl.BlockSpec((1, D), zero2),
        ],
        out_specs=pl.BlockSpec((block, D), lambda i: (i, 0)),
        out_shape=jax.ShapeDtypeStruct((M, D), jnp.float32),
        compiler_params=pltpu.CompilerParams(
            dimension_semantics=("arbitrary",)),
    )(x3d, mask2d, W_sgb, b_sgb.reshape(1, D), W1, b1.reshape(1, D // 2),
      ln_scale.reshape(1, D // 2), ln_bias.reshape(1, D // 2), W2,
      b2.reshape(1, D))


def kernel(x3d, unmasked_idx, masked_idx, W_sgb, b_sgb, W1, b1, ln_scale,
           ln_bias, W2, b2):
    mask = _build_mask(masked_idx, unmasked_idx)
    out = _fused(x3d, mask.reshape(M, 1), W_sgb, b_sgb, W1, b1, ln_scale,
                 ln_bias, W2, b2)
    return out.reshape(BEV_H, BEV_W, BEV_Z, D)


# speedup vs baseline: 1.6316x; 1.0223x over previous
import functools

import jax
import jax.numpy as jnp
from jax import lax
from jax.experimental import pallas as pl
from jax.experimental.pallas import tpu as pltpu
from jax.experimental.pallas import tpu_sc as plsc

BEV_H, BEV_W, BEV_Z, D = 128, 128, 16, 128
M = BEV_H * BEV_W * BEV_Z
NIDX = M // 2

NC, NS, L = 2, 16, 16
NW = NC * NS
CHUNK = 128
ROWS_PER_W = NIDX // (NW * CHUNK)


HALF = M // 2
PAD = HALF
SLICE = HALF // NS
N_CHUNKS = NIDX // (NS * CHUNK)


def _mask_body(midx_hbm, mask_hbm, idx_v, zeros_v, addr_v, val_v, shared):
    c = lax.axis_index("c")
    s = lax.axis_index("s")
    base = c * HALF

    pltpu.sync_copy(midx_hbm.at[pl.ds(s * N_CHUNKS, N_CHUNKS)], idx_v)
    def zbody(i, carry):
        zeros_v[pl.ds(i * L, L)] = jnp.zeros((L,), jnp.float32)
        return carry
    lax.fori_loop(0, SLICE // L, zbody, 0)
    pltpu.sync_copy(zeros_v, shared.at[pl.ds(s * SLICE, SLICE)])
    plsc.subcore_barrier()

    def body(j, carry):
        for i in range(CHUNK // L):
            idx16 = idx_v[j, pl.ds(i * L, L)]
            loc = idx16 - base
            inr = (loc >= 0) & (loc < HALF)
            addr_v[pl.ds(i * L, L)] = jnp.where(inr, loc, PAD)
            val_v[pl.ds(i * L, L)] = jnp.where(inr, 1.0, 0.0)
        pltpu.sync_copy(val_v, shared.at[addr_v], add=True)
        return carry

    lax.fori_loop(0, N_CHUNKS, body, 0)
    plsc.subcore_barrier()
    pltpu.sync_copy(shared.at[pl.ds(s * SLICE, SLICE)],
                    mask_hbm.at[pl.ds(base + s * SLICE, SLICE)])


def _build_mask(masked_idx, unmasked_idx):
    del unmasked_idx
    midx = masked_idx.reshape(NS * N_CHUNKS, CHUNK)
    mesh = plsc.VectorSubcoreMesh(core_axis_name="c", subcore_axis_name="s")
    fn = functools.partial(
        pl.kernel,
        mesh=mesh,
        out_type=jax.ShapeDtypeStruct((M,), jnp.float32),
        scratch_types=[
            pltpu.VMEM((N_CHUNKS, CHUNK), jnp.int32),
            pltpu.VMEM((SLICE,), jnp.float32),
            pltpu.VMEM((CHUNK,), jnp.int32),
            pltpu.VMEM((CHUNK,), jnp.float32),
            pltpu.VMEM_SHARED((HALF + CHUNK,), jnp.float32),
        ],
    )(_mask_body)
    return fn(midx)


def _fused_body(x_ref, m_ref, wsgb_ref, bsgb_ref, w1_ref, b1_ref, lns_ref,
                lnb_ref, w2_ref, b2_ref, o_ref):
    H = D // 2
    xt = x_ref[...].T
    sgb = jnp.dot(xt, wsgb_ref[...],
                  preferred_element_type=jnp.float32) + bsgb_ref[...]
    h = jnp.dot(xt, w1_ref[...],
                preferred_element_type=jnp.float32) + b1_ref[...]
    ones = jnp.ones((H, H), jnp.float32)
    mu = jnp.dot(h, ones, preferred_element_type=jnp.float32) * (1.0 / H)
    dh = h - mu
    var = jnp.dot(dh * dh, ones, preferred_element_type=jnp.float32) * (1.0 / H)
    hn = dh * lax.rsqrt(var + 1e-5) * lns_ref[...] + lnb_ref[...]
    hn = jnp.maximum(hn, 0.01 * hn)
    prior = jnp.dot(hn, w2_ref[...],
                    preferred_element_type=jnp.float32) + b2_ref[...]
    o_ref[...] = jnp.where(m_ref[...] > 0.5, prior, sgb)


def _fused(x3d, mask2d, W_sgb, b_sgb, W1, b1, ln_scale, ln_bias, W2, b2,
           block=2048):
    grid = (M // block,)
    zero2 = lambda i: (0, 0)
    return pl.pallas_call(
        _fused_body,
        grid=grid,
        in_specs=[
            pl.BlockSpec((D, block), lambda i: (0, i)),
            pl.BlockSpec((block, 1), lambda i: (i, 0)),
            pl.BlockSpec((D, D), zero2),
            pl.BlockSpec((1, D), zero2),
            pl.BlockSpec((D, D // 2), zero2),
            pl.BlockSpec((1, D // 2), zero2),
            pl.BlockSpec((1, D // 2), zero2),
            pl.BlockSpec((1, D // 2), zero2),
            pl.BlockSpec((D // 2, D), zero2),
            pl.BlockSpec((1, D), zero2),
        ],
        out_specs=pl.BlockSpec((block, D), lambda i: (i, 0)),
        out_shape=jax.ShapeDtypeStruct((M, D), jnp.float32),
        compiler_params=pltpu.CompilerParams(
            dimension_semantics=("arbitrary",)),
    )(x3d, mask2d, W_sgb, b_sgb.reshape(1, D), W1, b1.reshape(1, D // 2),
      ln_scale.reshape(1, D // 2), ln_bias.reshape(1, D // 2), W2,
      b2.reshape(1, D))


def kernel(x3d, unmasked_idx, masked_idx, W_sgb, b_sgb, W1, b1, ln_scale,
           ln_bias, W2, b2):
    mask = _build_mask(masked_idx, unmasked_idx)
    out = _fused(x3d, mask.reshape(M, 1), W_sgb, b_sgb, W1, b1, ln_scale,
                 ln_bias, W2, b2)
    return out.reshape(BEV_H, BEV_W, BEV_Z, D)


# speedup vs baseline: 1.6332x; 1.0010x over previous
import functools

import jax
import jax.numpy as jnp
from jax import lax
from jax.experimental import pallas as pl
from jax.experimental.pallas import tpu as pltpu
from jax.experimental.pallas import tpu_sc as plsc

BEV_H, BEV_W, BEV_Z, D = 128, 128, 16, 128
M = BEV_H * BEV_W * BEV_Z
NIDX = M // 2

NC, NS, L = 2, 16, 16
NW = NC * NS
CHUNK = 128
ROWS_PER_W = NIDX // (NW * CHUNK)


HALF = M // 2
PAD = HALF
SLICE = HALF // NS
N_CHUNKS = NIDX // (NS * CHUNK)


RING = 4


def _mask_body(midx_hbm, mask_hbm, idx_v, zeros_v, addr_v, val_v, shared, sem):
    c = lax.axis_index("c")
    s = lax.axis_index("s")
    base = c * HALF

    pltpu.sync_copy(midx_hbm.at[pl.ds(s * N_CHUNKS, N_CHUNKS)], idx_v)
    def zbody(i, carry):
        zeros_v[pl.ds(i * L, L)] = jnp.zeros((L,), jnp.float32)
        return carry
    lax.fori_loop(0, SLICE // L, zbody, 0)
    pltpu.sync_copy(zeros_v, shared.at[pl.ds(s * SLICE, SLICE)])
    plsc.subcore_barrier()

    def body(j, carry):
        slot = lax.rem(j, RING)

        @pl.when(j >= RING)
        def _():
            pltpu.make_async_copy(
                val_v.at[slot], shared.at[addr_v.at[slot]], sem).wait()

        for i in range(CHUNK // L):
            idx16 = idx_v[j, pl.ds(i * L, L)]
            loc = idx16 - base
            inr = (loc >= 0) & (loc < HALF)
            addr_v[slot, pl.ds(i * L, L)] = jnp.where(inr, loc, PAD)
            val_v[slot, pl.ds(i * L, L)] = jnp.where(inr, 1.0, 0.0)
        pltpu.async_copy(val_v.at[slot], shared.at[addr_v.at[slot]], sem,
                         add=True)
        return carry

    lax.fori_loop(0, N_CHUNKS, body, 0)
    for k in range(RING):
        pltpu.make_async_copy(val_v.at[k], shared.at[addr_v.at[k]], sem).wait()
    plsc.subcore_barrier()
    pltpu.sync_copy(shared.at[pl.ds(s * SLICE, SLICE)],
                    mask_hbm.at[pl.ds(base + s * SLICE, SLICE)])


def _build_mask(masked_idx, unmasked_idx):
    del unmasked_idx
    midx = masked_idx.reshape(NS * N_CHUNKS, CHUNK)
    mesh = plsc.VectorSubcoreMesh(core_axis_name="c", subcore_axis_name="s")
    fn = functools.partial(
        pl.kernel,
        mesh=mesh,
        out_type=jax.ShapeDtypeStruct((M,), jnp.float32),
        scratch_types=[
            pltpu.VMEM((N_CHUNKS, CHUNK), jnp.int32),
            pltpu.VMEM((SLICE,), jnp.float32),
            pltpu.VMEM((RING, CHUNK), jnp.int32),
            pltpu.VMEM((RING, CHUNK), jnp.float32),
            pltpu.VMEM_SHARED((HALF + CHUNK,), jnp.float32),
            pltpu.SemaphoreType.DMA,
        ],
    )(_mask_body)
    return fn(midx)


def _fused_body(x_ref, m_ref, wsgb_ref, bsgb_ref, w1_ref, b1_ref, lns_ref,
                lnb_ref, w2_ref, b2_ref, o_ref):
    H = D // 2
    xt = x_ref[...].T
    sgb = jnp.dot(xt, wsgb_ref[...],
                  preferred_element_type=jnp.float32) + bsgb_ref[...]
    h = jnp.dot(xt, w1_ref[...],
                preferred_element_type=jnp.float32) + b1_ref[...]
    ones = jnp.ones((H, H), jnp.float32)
    mu = jnp.dot(h, ones, preferred_element_type=jnp.float32) * (1.0 / H)
    dh = h - mu
    var = jnp.dot(dh * dh, ones, preferred_element_type=jnp.float32) * (1.0 / H)
    hn = dh * lax.rsqrt(var + 1e-5) * lns_ref[...] + lnb_ref[...]
    hn = jnp.maximum(hn, 0.01 * hn)
    prior = jnp.dot(hn, w2_ref[...],
                    preferred_element_type=jnp.float32) + b2_ref[...]
    o_ref[...] = jnp.where(m_ref[...] > 0.5, prior, sgb)


def _fused(x3d, mask2d, W_sgb, b_sgb, W1, b1, ln_scale, ln_bias, W2, b2,
           block=2048):
    grid = (M // block,)
    zero2 = lambda i: (0, 0)
    return pl.pallas_call(
        _fused_body,
        grid=grid,
        in_specs=[
            pl.BlockSpec((D, block), lambda i: (0, i)),
            pl.BlockSpec((block, 1), lambda i: (i, 0)),
            pl.BlockSpec((D, D), zero2),
            pl.BlockSpec((1, D), zero2),
            pl.BlockSpec((D, D // 2), zero2),
            pl.BlockSpec((1, D // 2), zero2),
            pl.BlockSpec((1, D // 2), zero2),
            pl.BlockSpec((1, D // 2), zero2),
            pl.BlockSpec((D // 2, D), zero2),
            pl.BlockSpec((1, D), zero2),
        ],
        out_specs=pl.BlockSpec((block, D), lambda i: (i, 0)),
        out_shape=jax.ShapeDtypeStruct((M, D), jnp.float32),
        compiler_params=pltpu.CompilerParams(
            dimension_semantics=("arbitrary",)),
    )(x3d, mask2d, W_sgb, b_sgb.reshape(1, D), W1, b1.reshape(1, D // 2),
      ln_scale.reshape(1, D // 2), ln_bias.reshape(1, D // 2), W2,
      b2.reshape(1, D))


def kernel(x3d, unmasked_idx, masked_idx, W_sgb, b_sgb, W1, b1, ln_scale,
           ln_bias, W2, b2):
    mask = _build_mask(masked_idx, unmasked_idx)
    out = _fused(x3d, mask.reshape(M, 1), W_sgb, b_sgb, W1, b1, ln_scale,
                 ln_bias, W2, b2)
    return out.reshape(BEV_H, BEV_W, BEV_Z, D)


# speedup vs baseline: 1.9459x; 1.1915x over previous
import functools

import jax
import jax.numpy as jnp
from jax import lax
from jax.experimental import pallas as pl
from jax.experimental.pallas import tpu as pltpu
from jax.experimental.pallas import tpu_sc as plsc

BEV_H, BEV_W, BEV_Z, D = 128, 128, 16, 128
M = BEV_H * BEV_W * BEV_Z
NIDX = M // 2

NC, NS, L = 2, 16, 16
NW = NC * NS
CHUNK = 128
ROWS_PER_W = NIDX // (NW * CHUNK)


HALF = M // 2
PAD = HALF
SLICE = HALF // NS
N_CHUNKS = NIDX // (NS * CHUNK)


def _mask_body(midx_hbm, mask_hbm, idx_v, zeros_v, addr_v, val_v, shared):
    c = lax.axis_index("c")
    s = lax.axis_index("s")
    base = c * HALF

    pltpu.sync_copy(midx_hbm.at[pl.ds(s * N_CHUNKS, N_CHUNKS)], idx_v)
    def zbody(i, carry):
        zeros_v[pl.ds(i * L, L)] = jnp.zeros((L,), jnp.float32)
        return carry
    lax.fori_loop(0, SLICE // L, zbody, 0)
    pltpu.sync_copy(zeros_v, shared.at[pl.ds(s * SLICE, SLICE)])
    plsc.subcore_barrier()

    def body(j, carry):
        for i in range(CHUNK // L):
            idx16 = idx_v[j, pl.ds(i * L, L)]
            loc = idx16 - base
            inr = (loc >= 0) & (loc < HALF)
            addr_v[pl.ds(i * L, L)] = jnp.where(inr, loc, PAD)
            val_v[pl.ds(i * L, L)] = jnp.where(inr, 1.0, 0.0)
        pltpu.sync_copy(val_v, shared.at[addr_v], add=True)
        return carry

    lax.fori_loop(0, N_CHUNKS, body, 0)
    plsc.subcore_barrier()
    pltpu.sync_copy(shared.at[pl.ds(s * SLICE, SLICE)],
                    mask_hbm.at[pl.ds(base + s * SLICE, SLICE)])


def _build_mask(masked_idx, unmasked_idx):
    del unmasked_idx
    midx = masked_idx.reshape(NS * N_CHUNKS, CHUNK)
    mesh = plsc.VectorSubcoreMesh(core_axis_name="c", subcore_axis_name="s")
    fn = functools.partial(
        pl.kernel,
        mesh=mesh,
        out_type=jax.ShapeDtypeStruct((M,), jnp.float32),
        scratch_types=[
            pltpu.VMEM((N_CHUNKS, CHUNK), jnp.int32),
            pltpu.VMEM((SLICE,), jnp.float32),
            pltpu.VMEM((CHUNK,), jnp.int32),
            pltpu.VMEM((CHUNK,), jnp.float32),
            pltpu.VMEM_SHARED((HALF + CHUNK,), jnp.float32),
        ],
    )(_mask_body)
    return fn(midx)


def _fused_body(x_ref, m_ref, wsgb_ref, bsgb_ref, w1_ref, b1_ref, lns_ref,
                lnb_ref, w2_ref, b2_ref, o_ref):
    H = D // 2
    xt = x_ref[...].T
    sgb = jnp.dot(xt, wsgb_ref[...],
                  preferred_element_type=jnp.float32) + bsgb_ref[...]
    h = jnp.dot(xt, w1_ref[...],
                preferred_element_type=jnp.float32) + b1_ref[...]
    ones = jnp.ones((H, H), jnp.float32)
    mu = jnp.dot(h, ones, preferred_element_type=jnp.float32) * (1.0 / H)
    dh = h - mu
    var = jnp.dot(dh * dh, ones, preferred_element_type=jnp.float32) * (1.0 / H)
    hn = dh * lax.rsqrt(var + 1e-5) * lns_ref[...] + lnb_ref[...]
    hn = jnp.maximum(hn, 0.01 * hn)
    prior = jnp.dot(hn, w2_ref[...],
                    preferred_element_type=jnp.float32) + b2_ref[...]
    o_ref[...] = jnp.where(m_ref[...] > 0.5, prior, sgb)


def _fused(x3d, mask2d, W_sgb, b_sgb, W1, b1, ln_scale, ln_bias, W2, b2,
           block=2048):
    grid = (M // block,)
    zero2 = lambda i: (0, 0)
    return pl.pallas_call(
        _fused_body,
        grid=grid,
        in_specs=[
            pl.BlockSpec((D, block), lambda i: (0, i)),
            pl.BlockSpec((block, 1), lambda i: (i, 0)),
            pl.BlockSpec((D, D), zero2),
            pl.BlockSpec((1, D), zero2),
            pl.BlockSpec((D, D // 2), zero2),
            pl.BlockSpec((1, D // 2), zero2),
            pl.BlockSpec((1, D // 2), zero2),
            pl.BlockSpec((1, D // 2), zero2),
            pl.BlockSpec((D // 2, D), zero2),
            pl.BlockSpec((1, D), zero2),
        ],
        out_specs=pl.BlockSpec((block, D), lambda i: (i, 0)),
        out_shape=jax.ShapeDtypeStruct((M, D), jnp.float32),
        compiler_params=pltpu.CompilerParams(
            dimension_semantics=("arbitrary",)),
    )(x3d, mask2d, W_sgb, b_sgb.reshape(1, D), W1, b1.reshape(1, D // 2),
      ln_scale.reshape(1, D // 2), ln_bias.reshape(1, D // 2), W2,
      b2.reshape(1, D))


def kernel(x3d, unmasked_idx, masked_idx, W_sgb, b_sgb, W1, b1, ln_scale,
           ln_bias, W2, b2):
    mask = _build_mask(masked_idx, unmasked_idx)
    out = _fused(x3d, mask.reshape(M, 1), W_sgb, b_sgb, W1, b1, ln_scale,
                 ln_bias, W2, b2, block=8192)
    return out.reshape(BEV_H, BEV_W, BEV_Z, D)


# speedup vs baseline: 2.4562x; 1.2623x over previous
import functools

import jax
import jax.numpy as jnp
from jax import lax
from jax.experimental import pallas as pl
from jax.experimental.pallas import tpu as pltpu
from jax.experimental.pallas import tpu_sc as plsc

BEV_H, BEV_W, BEV_Z, D = 128, 128, 16, 128
M = BEV_H * BEV_W * BEV_Z
NIDX = M // 2

NC, NS, L = 2, 16, 16
NW = NC * NS
CHUNK = 128
ROWS_PER_W = NIDX // (NW * CHUNK)


HALF = M // 2
PAD = HALF
SLICE = HALF // NS
N_CHUNKS = NIDX // (NS * CHUNK)


def _mask_body(midx_hbm, mask_hbm, idx_v, zeros_v, addr_v, val_v, shared):
    c = lax.axis_index("c")
    s = lax.axis_index("s")
    base = c * HALF

    pltpu.sync_copy(midx_hbm.at[pl.ds(s * N_CHUNKS, N_CHUNKS)], idx_v)
    def zbody(i, carry):
        zeros_v[pl.ds(i * L, L)] = jnp.zeros((L,), jnp.float32)
        return carry
    lax.fori_loop(0, SLICE // L, zbody, 0)
    pltpu.sync_copy(zeros_v, shared.at[pl.ds(s * SLICE, SLICE)])
    plsc.subcore_barrier()

    def body(j, carry):
        for i in range(CHUNK // L):
            idx16 = idx_v[j, pl.ds(i * L, L)]
            loc = idx16 - base
            inr = (loc >= 0) & (loc < HALF)
            pad16 = PAD + i * L + jnp.arange(L, dtype=jnp.int32)
            addr_v[pl.ds(i * L, L)] = jnp.where(inr, loc, pad16)
            val_v[pl.ds(i * L, L)] = jnp.where(inr, 1.0, 0.0)
        pltpu.sync_copy(val_v, shared.at[addr_v], add=True)
        return carry

    lax.fori_loop(0, N_CHUNKS, body, 0)
    plsc.subcore_barrier()
    pltpu.sync_copy(shared.at[pl.ds(s * SLICE, SLICE)],
                    mask_hbm.at[pl.ds(base + s * SLICE, SLICE)])


def _build_mask(masked_idx, unmasked_idx):
    del unmasked_idx
    midx = masked_idx.reshape(NS * N_CHUNKS, CHUNK)
    mesh = plsc.VectorSubcoreMesh(core_axis_name="c", subcore_axis_name="s")
    fn = functools.partial(
        pl.kernel,
        mesh=mesh,
        out_type=jax.ShapeDtypeStruct((M,), jnp.float32),
        scratch_types=[
            pltpu.VMEM((N_CHUNKS, CHUNK), jnp.int32),
            pltpu.VMEM((SLICE,), jnp.float32),
            pltpu.VMEM((CHUNK,), jnp.int32),
            pltpu.VMEM((CHUNK,), jnp.float32),
            pltpu.VMEM_SHARED((HALF + CHUNK,), jnp.float32),
        ],
    )(_mask_body)
    return fn(midx)


def _fused_body(x_ref, m_ref, wsgb_ref, bsgb_ref, w1_ref, b1_ref, lns_ref,
                lnb_ref, w2_ref, b2_ref, o_ref):
    H = D // 2
    xt = x_ref[...].T
    sgb = jnp.dot(xt, wsgb_ref[...],
                  preferred_element_type=jnp.float32) + bsgb_ref[...]
    h = jnp.dot(xt, w1_ref[...],
                preferred_element_type=jnp.float32) + b1_ref[...]
    ones = jnp.ones((H, H), jnp.float32)
    mu = jnp.dot(h, ones, preferred_element_type=jnp.float32) * (1.0 / H)
    dh = h - mu
    var = jnp.dot(dh * dh, ones, preferred_element_type=jnp.float32) * (1.0 / H)
    hn = dh * lax.rsqrt(var + 1e-5) * lns_ref[...] + lnb_ref[...]
    hn = jnp.maximum(hn, 0.01 * hn)
    prior = jnp.dot(hn, w2_ref[...],
                    preferred_element_type=jnp.float32) + b2_ref[...]
    o_ref[...] = jnp.where(m_ref[...] > 0.5, prior, sgb)


def _fused(x3d, mask2d, W_sgb, b_sgb, W1, b1, ln_scale, ln_bias, W2, b2,
           block=2048):
    grid = (M // block,)
    zero2 = lambda i: (0, 0)
    return pl.pallas_call(
        _fused_body,
        grid=grid,
        in_specs=[
            pl.BlockSpec((D, block), lambda i: (0, i)),
            pl.BlockSpec((block, 1), lambda i: (i, 0)),
            pl.BlockSpec((D, D), zero2),
            pl.BlockSpec((1, D), zero2),
            pl.BlockSpec((D, D // 2), zero2),
            pl.BlockSpec((1, D // 2), zero2),
            pl.BlockSpec((1, D // 2), zero2),
            pl.BlockSpec((1, D // 2), zero2),
            pl.BlockSpec((D // 2, D), zero2),
            pl.BlockSpec((1, D), zero2),
        ],
        out_specs=pl.BlockSpec((block, D), lambda i: (i, 0)),
        out_shape=jax.ShapeDtypeStruct((M, D), jnp.float32),
        compiler_params=pltpu.CompilerParams(
            dimension_semantics=("arbitrary",)),
    )(x3d, mask2d, W_sgb, b_sgb.reshape(1, D), W1, b1.reshape(1, D // 2),
      ln_scale.reshape(1, D // 2), ln_bias.reshape(1, D // 2), W2,
      b2.reshape(1, D))


def kernel(x3d, unmasked_idx, masked_idx, W_sgb, b_sgb, W1, b1, ln_scale,
           ln_bias, W2, b2):
    mask = _build_mask(masked_idx, unmasked_idx)
    out = _fused(x3d, mask.reshape(M, 1), W_sgb, b_sgb, W1, b1, ln_scale,
                 ln_bias, W2, b2, block=8192)
    return out.reshape(BEV_H, BEV_W, BEV_Z, D)
